# Initial kernel scaffold; baseline (speedup 1.0000x reference)
import jax
import jax.numpy as jnp
from jax.experimental import pallas as pl


def kernel(mu, log_sigma, noise, family_ids, batch_size):
    raise NotImplementedError("write your pallas kernel here")


# speedup vs baseline: 1.2849x; 1.2849x over previous
import functools

import jax
import jax.numpy as jnp
from jax import lax
from jax.experimental import pallas as pl
from jax.experimental.pallas import tpu as pltpu
from jax.experimental.pallas import tpu_sc as plsc

_NUM_CORES = 2
_NUM_SUBCORES = 16
_NUM_WORKERS = _NUM_CORES * _NUM_SUBCORES
_LANES = 16
_LN10 = 2.302585092994045684


@functools.lru_cache(maxsize=None)
def _build(n_families: int, batch: int):
    assert batch % (8 * _NUM_WORKERS) == 0
    b_per_w = batch // _NUM_WORKERS
    mesh = plsc.VectorSubcoreMesh(core_axis_name="c", subcore_axis_name="s")

    @functools.partial(
        pl.kernel,
        mesh=mesh,
        out_type=jax.ShapeDtypeStruct((batch,), jnp.float32),
        scratch_types=[
            pltpu.VMEM((b_per_w,), jnp.int32),
            pltpu.VMEM((b_per_w,), jnp.float32),
            pltpu.VMEM((b_per_w,), jnp.float32),
            pltpu.VMEM((b_per_w,), jnp.float32),
            pltpu.SemaphoreType.DMA,
        ],
    )
    def sample_kernel(mu_hbm, ls_hbm, noise_hbm, ids_hbm, out_hbm,
                      idx_v, mu_v, sig_v, noise_v, sem):
        wid = lax.axis_index("s") * _NUM_CORES + lax.axis_index("c")
        base = wid * b_per_w
        pltpu.sync_copy(ids_hbm.at[pl.ds(base, b_per_w)], idx_v)
        gat_mu = pltpu.async_copy(mu_hbm.at[idx_v], mu_v, sem)
        gat_ls = pltpu.async_copy(ls_hbm.at[idx_v], sig_v, sem)
        pltpu.sync_copy(noise_hbm.at[pl.ds(base, b_per_w)], noise_v)
        gat_mu.wait()
        gat_ls.wait()
        for i in range(b_per_w // _LANES):
            s = pl.ds(i * _LANES, _LANES)
            m = mu_v[s]
            g = sig_v[s]
            n = noise_v[s]
            mu_v[s] = jnp.exp(_LN10 * m + (_LN10 * jnp.exp(g)) * n)
        pltpu.sync_copy(mu_v, out_hbm.at[pl.ds(base, b_per_w)])

    return sample_kernel


def kernel(mu, log_sigma, noise, family_ids, batch_size):
    del batch_size
    fn = _build(mu.shape[0], noise.shape[0])
    return fn(mu, log_sigma, noise, family_ids)


# speedup vs baseline: 1.3303x; 1.0353x over previous
import functools

import jax
import jax.numpy as jnp
from jax import lax
from jax.experimental import pallas as pl
from jax.experimental.pallas import tpu as pltpu
from jax.experimental.pallas import tpu_sc as plsc

_NUM_CORES = 2
_NUM_SUBCORES = 16
_NUM_WORKERS = _NUM_CORES * _NUM_SUBCORES
_LANES = 16
_LN10 = 2.302585092994045684


@functools.lru_cache(maxsize=None)
def _build(n_families: int, batch: int):
    assert batch % (8 * _NUM_WORKERS) == 0
    b_per_w = batch // _NUM_WORKERS
    mesh = plsc.VectorSubcoreMesh(core_axis_name="c", subcore_axis_name="s")

    @functools.partial(
        pl.kernel,
        mesh=mesh,
        out_type=jax.ShapeDtypeStruct((batch,), jnp.float32),
        scratch_types=[
            pltpu.VMEM((b_per_w,), jnp.int32),
            pltpu.VMEM((b_per_w,), jnp.float32),
            pltpu.VMEM((b_per_w,), jnp.float32),
            pltpu.VMEM((b_per_w,), jnp.float32),
            pltpu.SemaphoreType.DMA,
        ],
    )
    def sample_kernel(mu_hbm, ls_hbm, noise_hbm, ids_hbm, out_hbm,
                      idx_v, mu_v, sig_v, noise_v, sem):
        wid = lax.axis_index("s") * _NUM_CORES + lax.axis_index("c")
        base = wid * b_per_w
        pltpu.sync_copy(ids_hbm.at[pl.ds(base, b_per_w)], idx_v)
        gat_mu = pltpu.async_copy(mu_hbm.at[idx_v], mu_v, sem)
        gat_ls = pltpu.async_copy(ls_hbm.at[idx_v], sig_v, sem)
        pltpu.sync_copy(noise_hbm.at[pl.ds(base, b_per_w)], noise_v)
        gat_mu.wait()
        gat_ls.wait()
        def body(i, carry):
            s = pl.ds(i * _LANES, _LANES)
            m = mu_v[s]
            g = sig_v[s]
            n = noise_v[s]
            mu_v[s] = jnp.exp(_LN10 * m + (_LN10 * jnp.exp(g)) * n)
            return carry

        lax.fori_loop(0, b_per_w // _LANES, body, 0, unroll=False)
        pltpu.sync_copy(mu_v, out_hbm.at[pl.ds(base, b_per_w)])

    return sample_kernel


def kernel(mu, log_sigma, noise, family_ids, batch_size):
    del batch_size
    fn = _build(mu.shape[0], noise.shape[0])
    return fn(mu, log_sigma, noise, family_ids)
